# baseline (speedup 1.0000x reference)
import jax
import jax.numpy as jnp
from jax import lax
from jax.experimental import pallas as pl
from jax.experimental.pallas import tpu as pltpu
from jax.experimental.pallas import tpu_sc as plsc

_NC = 2
_NS = 16
_NW = _NC * _NS

_V = 1_000_000
_B = 16384 * 200
_D = 16
_CHUNK = 1280
_TPC = _CHUNK // 128 * 8
_NPH = 4
_BPW = _B // _NW
_BPP = _BPW // _NPH
_NCH = _BPP // _CHUNK


def _gather_body(table_hbm, idx_hbm, out_hbm,
                 ibuf, rows0, rows1, t0, t1,
                 gsem0, gsem1, wsem0, wsem1):
    row_bufs = (rows0, rows1)
    t_bufs = (t0, t1)
    gsems = (gsem0, gsem1)
    wsems = (wsem0, wsem1)
    wid = lax.axis_index("s") * _NC + lax.axis_index("c")
    base = wid * _BPW

    j16 = lax.iota(jnp.int32, 16)
    jrow_pat = (j16 >> 3) * _TPC + (j16 & 7)

    def start_gather(s, k):
        pltpu.async_copy(
            table_hbm.at[ibuf.at[pl.ds(k * _CHUNK, _CHUNK)]],
            row_bufs[s], gsems[s])

    def wait_gather(s):
        pltpu.make_async_copy(
            table_hbm.at[pl.ds(0, _CHUNK)], row_bufs[s], gsems[s]).wait()

    def wait_wb(s):
        for h in range(2):
            pltpu.make_async_copy(
                out_hbm.at[pl.ds(0, _TPC), :],
                t_bufs[s].at[pl.ds(h * _TPC, _TPC), :], wsems[s]).wait()

    @pl.loop(0, _NPH)
    def _phase(ph):
        pbase = base + ph * _BPP
        pltpu.sync_copy(idx_hbm.at[pl.ds(pbase, _BPP)], ibuf)
        start_gather(0, 0)
        start_gather(1, 1)

        @pl.loop(0, _NCH, step=2)
        def _pair(k0):
            for s in range(2):
                k = k0 + s
                wait_gather(s)

                @pl.when(k0 + s >= 2)
                def _():
                    wait_wb(s)

                rows = row_bufs[s]
                tbuf = t_bufs[s]

                @pl.loop(0, _CHUNK // 128)
                def _bt(bt):
                    rowvec = jrow_pat + bt * 8
                    for c0 in range(0, 128, 8):
                        vs = [rows[bt * 128 + c0 + i, :] for i in range(8)]
                        for i in range(8):
                            plsc.store_scatter(
                                tbuf,
                                [rowvec, jnp.full((16,), c0 + i, jnp.int32)],
                                vs[i])

                off = pbase + k * _CHUNK
                q0 = off // 128 * 8
                pltpu.async_copy(tbuf.at[pl.ds(0, _TPC), :],
                                 out_hbm.at[pl.ds(q0, _TPC), :], wsems[s])
                pltpu.async_copy(
                    tbuf.at[pl.ds(_TPC, _TPC), :],
                    out_hbm.at[pl.ds(_B // 128 * 8 + q0, _TPC), :], wsems[s])

                @pl.when(k + 2 < _NCH)
                def _():
                    start_gather(s, k + 2)

        wait_wb(0)
        wait_wb(1)


@jax.jit
def _gather(weights2d, idx):
    mesh = plsc.VectorSubcoreMesh(core_axis_name="c", subcore_axis_name="s")
    f = pl.kernel(
        _gather_body,
        out_type=jax.ShapeDtypeStruct((_B * _D // 128, 128), jnp.float32),
        mesh=mesh,
        scratch_types=[
            pltpu.VMEM((_BPP,), jnp.int32),
            pltpu.VMEM((_CHUNK, _D), jnp.float32),
            pltpu.VMEM((_CHUNK, _D), jnp.float32),
            pltpu.VMEM((2 * _TPC, 128), jnp.float32),
            pltpu.VMEM((2 * _TPC, 128), jnp.float32),
            pltpu.SemaphoreType.DMA,
            pltpu.SemaphoreType.DMA,
            pltpu.SemaphoreType.DMA,
            pltpu.SemaphoreType.DMA,
        ],
        compiler_params=pltpu.CompilerParams(
            use_tc_tiling_on_sc=False, needs_layout_passes=False),
    )
    return f(weights2d, idx)


def kernel(x, weights):
    idx = x[:, 2:].astype(jnp.int32).reshape(-1)
    out = _gather(weights, idx)
    mean = (out.reshape(2, _B // 128, 8, 128)
            .transpose(1, 3, 0, 2)
            .reshape(_B, _D))
    sigma = jnp.array([1.0], dtype=jnp.float32)
    return (mean, sigma)


# speedup vs baseline: 1.0094x; 1.0094x over previous
import jax
import jax.numpy as jnp
from jax import lax
from jax.experimental import pallas as pl
from jax.experimental.pallas import tpu as pltpu
from jax.experimental.pallas import tpu_sc as plsc

_NC = 2
_NS = 16
_NW = _NC * _NS

_V = 1_000_000
_B = 16384 * 200
_D = 16
_CHUNK = 1024
_TPC = _CHUNK // 128 * 8
_NPH = 2
_BPW = _B // _NW
_BPP = _BPW // _NPH
_NCH = _BPP // _CHUNK


def _gather_body(table_hbm, idx_hbm, out_hbm,
                 ibuf, rows0, rows1, t0, t1,
                 gsem0, gsem1, wsem0, wsem1):
    row_bufs = (rows0, rows1)
    t_bufs = (t0, t1)
    gsems = (gsem0, gsem1)
    wsems = (wsem0, wsem1)
    wid = lax.axis_index("s") * _NC + lax.axis_index("c")
    base = wid * _BPW

    j16 = lax.iota(jnp.int32, 16)
    jrow_pat = (j16 >> 3) * _TPC + (j16 & 7)

    def start_gather(s, k):
        half = _CHUNK // 2
        for h in range(2):
            pltpu.async_copy(
                table_hbm.at[ibuf.at[pl.ds(k * _CHUNK + h * half, half)]],
                row_bufs[s].at[pl.ds(h * half, half), :], gsems[s])

    def wait_gather(s):
        half = _CHUNK // 2
        for h in range(2):
            pltpu.make_async_copy(
                table_hbm.at[pl.ds(0, half)],
                row_bufs[s].at[pl.ds(h * half, half), :], gsems[s]).wait()

    def wait_wb(s):
        for h in range(2):
            pltpu.make_async_copy(
                out_hbm.at[pl.ds(0, _TPC), :],
                t_bufs[s].at[pl.ds(h * _TPC, _TPC), :], wsems[s]).wait()

    @pl.loop(0, _NPH)
    def _phase(ph):
        pbase = base + ph * _BPP
        pltpu.sync_copy(idx_hbm.at[pl.ds(pbase, _BPP)], ibuf)
        start_gather(0, 0)
        start_gather(1, 1)

        @pl.loop(0, _NCH, step=2)
        def _pair(k0):
            for s in range(2):
                k = k0 + s
                wait_gather(s)

                @pl.when(k0 + s >= 2)
                def _():
                    wait_wb(s)

                rows = row_bufs[s]
                tbuf = t_bufs[s]

                @pl.loop(0, _CHUNK // 128)
                def _bt(bt):
                    rowvec = jrow_pat + bt * 8
                    for c0 in range(0, 128, 8):
                        vs = [rows[bt * 128 + c0 + i, :] for i in range(8)]
                        for i in range(8):
                            plsc.store_scatter(
                                tbuf,
                                [rowvec, jnp.full((16,), c0 + i, jnp.int32)],
                                vs[i])

                off = pbase + k * _CHUNK
                q0 = off // 128 * 8
                pltpu.async_copy(tbuf.at[pl.ds(0, _TPC), :],
                                 out_hbm.at[pl.ds(q0, _TPC), :], wsems[s])
                pltpu.async_copy(
                    tbuf.at[pl.ds(_TPC, _TPC), :],
                    out_hbm.at[pl.ds(_B // 128 * 8 + q0, _TPC), :], wsems[s])

                @pl.when(k + 2 < _NCH)
                def _():
                    start_gather(s, k + 2)

        wait_wb(0)
        wait_wb(1)


@jax.jit
def _gather(weights2d, idx):
    mesh = plsc.VectorSubcoreMesh(core_axis_name="c", subcore_axis_name="s")
    f = pl.kernel(
        _gather_body,
        out_type=jax.ShapeDtypeStruct((_B * _D // 128, 128), jnp.float32),
        mesh=mesh,
        scratch_types=[
            pltpu.VMEM((_BPP,), jnp.int32),
            pltpu.VMEM((_CHUNK, _D), jnp.float32),
            pltpu.VMEM((_CHUNK, _D), jnp.float32),
            pltpu.VMEM((2 * _TPC, 128), jnp.float32),
            pltpu.VMEM((2 * _TPC, 128), jnp.float32),
            pltpu.SemaphoreType.DMA,
            pltpu.SemaphoreType.DMA,
            pltpu.SemaphoreType.DMA,
            pltpu.SemaphoreType.DMA,
        ],
        compiler_params=pltpu.CompilerParams(
            use_tc_tiling_on_sc=False, needs_layout_passes=False),
    )
    return f(weights2d, idx)


def kernel(x, weights):
    idx = x[:, 2:].astype(jnp.int32).reshape(-1)
    out = _gather(weights, idx)
    mean = (out.reshape(2, _B // 128, 8, 128)
            .transpose(1, 3, 0, 2)
            .reshape(_B, _D))
    sigma = jnp.array([1.0], dtype=jnp.float32)
    return (mean, sigma)
